# baseline (speedup 1.0000x reference)
import functools

import numpy as np

import jax
import jax.numpy as jnp
from jax import lax
from jax.experimental import pallas as pl
from jax.experimental.pallas import tpu as pltpu
from jax.experimental.pallas import tpu_sc as plsc

N = 10000
D = 128
E = 320000

NC = 2
NS = 16
NW = NC * NS

NP = 10240
STRIPE = NP // NS

B = 80
E2 = 330240
EPT = E2 // NW
NCHUNK = EPT // B
NBUF = 4

KB2 = 8
B2 = 125
NG2 = (E // NW) // (KB2 * B2)

_t = np.arange(D)
_u = _t % 32
_PI = (_t // 32) * 32 + np.where(_u < 16, 2 * _u, 2 * (_u - 16) + 1)
_PINV = np.argsort(_PI)

_mesh = plsc.VectorSubcoreMesh(core_axis_name="c", subcore_axis_name="s")


@functools.partial(
    pl.kernel,
    out_type=jax.ShapeDtypeStruct((NC, NP), jnp.float32),
    mesh=_mesh,
    scratch_types=[
        pltpu.VMEM((KB2, B2), jnp.int32),
        pltpu.VMEM((128,), jnp.float32),
        pltpu.VMEM((STRIPE,), jnp.float32),
        pltpu.VMEM_SHARED((NP,), jnp.float32),
        pltpu.SemaphoreType.DMA,
    ],
)
def _deg_kernel(dst3_hbm, out_hbm, didx_v, ones_v, zbuf_v, deg_sh, dsem):
    cid = lax.axis_index("c")
    sid = lax.axis_index("s")
    wid = sid * NC + cid

    for k in range(STRIPE // 16):
        zbuf_v[pl.ds(k * 16, 16)] = jnp.zeros((16,), jnp.float32)
    for k in range(128 // 16):
        ones_v[pl.ds(k * 16, 16)] = jnp.ones((16,), jnp.float32)

    pltpu.sync_copy(zbuf_v, deg_sh.at[pl.ds(sid * STRIPE, STRIPE)])
    plsc.subcore_barrier()

    def group(f, carry):
        pltpu.sync_copy(dst3_hbm.at[wid, pl.ds(f * KB2, KB2)], didx_v)
        for j in range(KB2):
            pltpu.async_copy(ones_v.at[pl.ds(0, B2)],
                             deg_sh.at[didx_v.at[j]], dsem, add=True)
        for j in range(KB2):
            pltpu.make_async_copy(ones_v.at[pl.ds(0, B2)],
                                  deg_sh.at[didx_v.at[0]], dsem).wait()
        return carry

    lax.fori_loop(0, NG2, group, 0)
    plsc.subcore_barrier()
    pltpu.sync_copy(deg_sh.at[pl.ds(sid * STRIPE, STRIPE)],
                    out_hbm.at[cid, pl.ds(sid * STRIPE, STRIPE)])


@functools.partial(
    pl.kernel,
    out_type=jax.ShapeDtypeStruct((NC, NP, D), jnp.float32),
    mesh=_mesh,
    compiler_params=pltpu.CompilerParams(needs_layout_passes=False,
                                         use_tc_tiling_on_sc=False),
    scratch_types=[
        pltpu.VMEM((16, B), jnp.int32),
        pltpu.VMEM((16, B), jnp.int32),
        pltpu.VMEM((NBUF, B, D // 2), jnp.int32),
        pltpu.VMEM((2, B, D), jnp.float32),
        pltpu.VMEM_SHARED((NP, D), jnp.float32),
        pltpu.SemaphoreType.DMA,
        pltpu.SemaphoreType.DMA,
    ],
)
def _msg_kernel(g_hbm, src3_hbm, dst3_hbm, zeros_hbm, out_hbm,
                sidx_g, didx_g, rows_i, conv_f, acc_sh, gsem, ssem0):
    cid = lax.axis_index("c")
    sid = lax.axis_index("s")
    wid = sid * NC + cid

    pltpu.sync_copy(zeros_hbm.at[pl.ds(sid * STRIPE, STRIPE)],
                    acc_sh.at[pl.ds(sid * STRIPE, STRIPE)])
    plsc.subcore_barrier()

    def fetch_group(f):
        p = lax.rem(f, 2)

        @pl.when(f < NCHUNK // 8)
        def _():
            pltpu.sync_copy(src3_hbm.at[wid, pl.ds(f * 8, 8)],
                            sidx_g.at[pl.ds(8 * p, 8)])
            pltpu.sync_copy(dst3_hbm.at[wid, pl.ds(f * 8, 8)],
                            didx_g.at[pl.ds(8 * p, 8)])

        @pl.when(f == NCHUNK // 8)
        def _():
            pltpu.sync_copy(src3_hbm.at[wid, pl.ds(128, NCHUNK - 128)],
                            sidx_g.at[pl.ds(8 * p, NCHUNK - 128)])
            pltpu.sync_copy(dst3_hbm.at[wid, pl.ds(128, NCHUNK - 128)],
                            didx_g.at[pl.ds(8 * p, NCHUNK - 128)])

    def idx_row(c):
        return 8 * lax.rem(lax.div(c, 8), 2) + lax.rem(c, 8)

    def gather(c):
        pltpu.async_copy(g_hbm.at[sidx_g.at[idx_row(c)]],
                         rows_i.at[lax.rem(c, NBUF)], gsem)

    def wait_gather(c):
        pltpu.make_async_copy(g_hbm.at[sidx_g.at[0]],
                              rows_i.at[lax.rem(c, NBUF)], gsem).wait()

    def convert(c):
        b = lax.rem(c, NBUF)
        fs = lax.rem(c, 2)

        def crow(i, carry):
            for j in range(D // 32):
                v = rows_i[b, i, pl.ds(16 * j, 16)]
                lo = plsc.bitcast(jnp.left_shift(v, 16), jnp.float32)
                hi = plsc.bitcast(
                    jnp.bitwise_and(v, jnp.int32(-65536)), jnp.float32)
                conv_f[fs, i, pl.ds(32 * j, 16)] = lo
                conv_f[fs, i, pl.ds(32 * j + 16, 16)] = hi
            return carry

        lax.fori_loop(0, B, crow, 0)

    def scatter(c, sem):
        pltpu.async_copy(conv_f.at[lax.rem(c, 2)],
                         acc_sh.at[didx_g.at[idx_row(c)]], sem, add=True)

    def drain_scatter(sem):
        pltpu.make_async_copy(conv_f.at[0], acc_sh.at[didx_g.at[0]],
                              sem).wait()

    fetch_group(0)
    gather(0)
    gather(1)
    gather(2)

    def step(c, carry):
        wait_gather(c)
        convert(c)

        @pl.when(c >= 1)
        def _():
            drain_scatter(ssem0)

        scatter(c, ssem0)

        @pl.when(c + 3 < NCHUNK)
        def _():
            @pl.when(lax.rem(c + 3, 8) == 0)
            def _():
                fetch_group(lax.div(c + 3, 8))

            gather(c + 3)

        return carry

    lax.fori_loop(0, NCHUNK, step, 0)
    drain_scatter(ssem0)
    plsc.subcore_barrier()
    pltpu.sync_copy(acc_sh.at[pl.ds(sid * STRIPE, STRIPE)],
                    out_hbm.at[cid, pl.ds(sid * STRIPE, STRIPE)])


RB = 5120
RBP = 5000


def _dinv_block(degp):
    deg = jnp.sum(degp, axis=1) + 1.0
    return lax.rsqrt(deg)


def _tc_pre_body(x_ref, w_ref, degp_ref, g_ref):
    dinv = _dinv_block(degp_ref[...])
    h = jnp.dot(x_ref[...], w_ref[...], preferred_element_type=jnp.float32)
    g_ref[...] = (h * dinv[:, None]).astype(jnp.bfloat16)


def _tc_mid_body(p_ref, degp_ref, b_ref, w_ref, g2_ref):
    dinv = _dinv_block(degp_ref[...])
    s = p_ref[0] + p_ref[1]
    z = jnp.maximum(s * dinv[:, None] + b_ref[...], 0.0)
    h2 = jnp.dot(z, w_ref[...], preferred_element_type=jnp.float32)
    g2_ref[...] = (h2 * dinv[:, None]).astype(jnp.bfloat16)


def _tc_post_body(p_ref, degp_ref, b_ref, out_ref):
    dinv = _dinv_block(degp_ref[...])
    s = p_ref[0] + p_ref[1]
    out_ref[...] = s * dinv[:, None] + b_ref[...]


_x_spec = pl.BlockSpec((RB, D), lambda i: (i, 0))
_w_spec = pl.BlockSpec((D, D), lambda i: (0, 0))
_degp_spec = pl.BlockSpec((RB, NC), lambda i: (i, 0))
_p_spec = pl.BlockSpec((NC, RB, D), lambda i: (0, i, 0))
_b_spec = pl.BlockSpec((1, D), lambda i: (0, 0))

_tc_pre = pl.pallas_call(
    _tc_pre_body, grid=(NP // RB,),
    in_specs=[_x_spec, _w_spec, _degp_spec],
    out_specs=pl.BlockSpec((RB, D), lambda i: (i, 0)),
    out_shape=jax.ShapeDtypeStruct((NP, D), jnp.bfloat16))

_tc_mid = pl.pallas_call(
    _tc_mid_body, grid=(NP // RB,),
    in_specs=[_p_spec, _degp_spec, _b_spec, _w_spec],
    out_specs=pl.BlockSpec((RB, D), lambda i: (i, 0)),
    out_shape=jax.ShapeDtypeStruct((NP, D), jnp.bfloat16))

_tc_post = pl.pallas_call(
    _tc_post_body, grid=(N // RBP,),
    in_specs=[pl.BlockSpec((NC, RBP, D), lambda i: (0, i, 0)),
              pl.BlockSpec((RBP, NC), lambda i: (i, 0)),
              _b_spec],
    out_specs=pl.BlockSpec((RBP, D), lambda i: (i, 0)),
    out_shape=jax.ShapeDtypeStruct((N, D), jnp.float32))


def _as_i32_pairs(g_bf):
    return lax.bitcast_convert_type(g_bf.reshape(NP, D // 2, 2), jnp.int32)


@jax.jit
def kernel(x, edge_index, batch, W1, b1, W2, b2):
    src = edge_index[0]
    dst = edge_index[1]
    loop = jnp.arange(N, dtype=jnp.int32)
    pad = jnp.full((E2 - E - N,), N, jnp.int32)
    src3 = jnp.concatenate([src, loop, pad]).reshape(NW, NCHUNK, B)
    dst3 = jnp.concatenate([dst, loop, pad]).reshape(NW, NCHUNK, B)
    zeros = jnp.zeros((NP, D), jnp.float32)
    xp = jnp.concatenate([x, jnp.zeros((NP - N, D), jnp.float32)])
    W1p = W1[:, _PINV]
    W2p = W2[:, _PINV]
    b1r = b1.reshape(1, D)
    b2r = b2.reshape(1, D)

    degp = _deg_kernel(dst.reshape(NW, (E // NW) // B2, B2)).T
    g1 = _tc_pre(xp, W1p, degp)
    p1 = _msg_kernel(_as_i32_pairs(g1), src3, dst3, zeros)
    g2 = _tc_mid(p1, degp, b1r, W2p)
    p2 = _msg_kernel(_as_i32_pairs(g2), src3, dst3, zeros)
    out = _tc_post(p2, degp, b2r)
    return out


# speedup vs baseline: 1.0723x; 1.0723x over previous
import functools

import numpy as np

import jax
import jax.numpy as jnp
from jax import lax
from jax.experimental import pallas as pl
from jax.experimental.pallas import tpu as pltpu
from jax.experimental.pallas import tpu_sc as plsc

N = 10000
D = 128
E = 320000

NC = 2
NS = 16
NW = NC * NS

NP = 10240
STRIPE = NP // NS

B = 80
E2 = 330240
EPT = E2 // NW
NCHUNK = EPT // B
NBUF = 4

KB2 = 8
B2 = 125
NG2 = (E // NW) // (KB2 * B2)

_t = np.arange(D)
_u = _t % 32
_PI = (_t // 32) * 32 + np.where(_u < 16, 2 * _u, 2 * (_u - 16) + 1)
_PINV = np.argsort(_PI)

_mesh = plsc.VectorSubcoreMesh(core_axis_name="c", subcore_axis_name="s")


@functools.partial(
    pl.kernel,
    out_type=jax.ShapeDtypeStruct((NC, NP), jnp.float32),
    mesh=_mesh,
    scratch_types=[
        pltpu.VMEM((KB2, B2), jnp.int32),
        pltpu.VMEM((128,), jnp.float32),
        pltpu.VMEM((STRIPE,), jnp.float32),
        pltpu.VMEM_SHARED((NP,), jnp.float32),
        pltpu.SemaphoreType.DMA,
    ],
)
def _deg_kernel(dst3_hbm, out_hbm, didx_v, ones_v, zbuf_v, deg_sh, dsem):
    cid = lax.axis_index("c")
    sid = lax.axis_index("s")
    wid = sid * NC + cid

    for k in range(STRIPE // 16):
        zbuf_v[pl.ds(k * 16, 16)] = jnp.zeros((16,), jnp.float32)
    for k in range(128 // 16):
        ones_v[pl.ds(k * 16, 16)] = jnp.ones((16,), jnp.float32)

    pltpu.sync_copy(zbuf_v, deg_sh.at[pl.ds(sid * STRIPE, STRIPE)])
    plsc.subcore_barrier()

    def group(f, carry):
        pltpu.sync_copy(dst3_hbm.at[wid, pl.ds(f * KB2, KB2)], didx_v)
        for j in range(KB2):
            pltpu.async_copy(ones_v.at[pl.ds(0, B2)],
                             deg_sh.at[didx_v.at[j]], dsem, add=True)
        for j in range(KB2):
            pltpu.make_async_copy(ones_v.at[pl.ds(0, B2)],
                                  deg_sh.at[didx_v.at[0]], dsem).wait()
        return carry

    lax.fori_loop(0, NG2, group, 0)
    plsc.subcore_barrier()
    pltpu.sync_copy(deg_sh.at[pl.ds(sid * STRIPE, STRIPE)],
                    out_hbm.at[cid, pl.ds(sid * STRIPE, STRIPE)])


@functools.partial(
    pl.kernel,
    out_type=jax.ShapeDtypeStruct((NC, NP, D), jnp.float32),
    mesh=_mesh,
    compiler_params=pltpu.CompilerParams(needs_layout_passes=False,
                                         use_tc_tiling_on_sc=False),
    scratch_types=[
        pltpu.VMEM((16, B), jnp.int32),
        pltpu.VMEM((16, B), jnp.int32),
        pltpu.VMEM((NBUF, B, D // 2), jnp.int32),
        pltpu.VMEM((2, B, D), jnp.float32),
        pltpu.VMEM_SHARED((NP, D), jnp.float32),
        pltpu.SemaphoreType.DMA,
        pltpu.SemaphoreType.DMA,
    ],
)
def _msg_kernel(g_hbm, src3_hbm, dst3_hbm, zeros_hbm, out_hbm,
                sidx_g, didx_g, rows_i, conv_f, acc_sh, gsem, ssem0):
    cid = lax.axis_index("c")
    sid = lax.axis_index("s")
    wid = sid * NC + cid

    pltpu.sync_copy(zeros_hbm.at[pl.ds(sid * STRIPE, STRIPE)],
                    acc_sh.at[pl.ds(sid * STRIPE, STRIPE)])
    plsc.subcore_barrier()

    def fetch_group(f):
        p = lax.rem(f, 2)

        @pl.when(f < NCHUNK // 8)
        def _():
            pltpu.sync_copy(src3_hbm.at[wid, pl.ds(f * 8, 8)],
                            sidx_g.at[pl.ds(8 * p, 8)])
            pltpu.sync_copy(dst3_hbm.at[wid, pl.ds(f * 8, 8)],
                            didx_g.at[pl.ds(8 * p, 8)])

        @pl.when(f == NCHUNK // 8)
        def _():
            pltpu.sync_copy(src3_hbm.at[wid, pl.ds(128, NCHUNK - 128)],
                            sidx_g.at[pl.ds(8 * p, NCHUNK - 128)])
            pltpu.sync_copy(dst3_hbm.at[wid, pl.ds(128, NCHUNK - 128)],
                            didx_g.at[pl.ds(8 * p, NCHUNK - 128)])

    def idx_row(c):
        return 8 * lax.rem(lax.div(c, 8), 2) + lax.rem(c, 8)

    def gather(c):
        pltpu.async_copy(g_hbm.at[sidx_g.at[idx_row(c)]],
                         rows_i.at[lax.rem(c, NBUF)], gsem)

    def wait_gather(c):
        pltpu.make_async_copy(g_hbm.at[sidx_g.at[0]],
                              rows_i.at[lax.rem(c, NBUF)], gsem).wait()

    def convert(c):
        b = lax.rem(c, NBUF)
        fs = lax.rem(c, 2)

        for i in range(B):
            for j in range(D // 32):
                v = rows_i[b, i, pl.ds(16 * j, 16)]
                lo = plsc.bitcast(jnp.left_shift(v, 16), jnp.float32)
                hi = plsc.bitcast(
                    jnp.bitwise_and(v, jnp.int32(-65536)), jnp.float32)
                conv_f[fs, i, pl.ds(32 * j, 16)] = lo
                conv_f[fs, i, pl.ds(32 * j + 16, 16)] = hi

    def scatter(c, sem):
        pltpu.async_copy(conv_f.at[lax.rem(c, 2)],
                         acc_sh.at[didx_g.at[idx_row(c)]], sem, add=True)

    def drain_scatter(sem):
        pltpu.make_async_copy(conv_f.at[0], acc_sh.at[didx_g.at[0]],
                              sem).wait()

    fetch_group(0)
    gather(0)
    gather(1)
    gather(2)

    def step(c, carry):
        wait_gather(c)
        convert(c)

        @pl.when(c >= 1)
        def _():
            drain_scatter(ssem0)

        scatter(c, ssem0)

        @pl.when(c + 3 < NCHUNK)
        def _():
            @pl.when(lax.rem(c + 3, 8) == 0)
            def _():
                fetch_group(lax.div(c + 3, 8))

            gather(c + 3)

        return carry

    lax.fori_loop(0, NCHUNK, step, 0)
    drain_scatter(ssem0)
    plsc.subcore_barrier()
    pltpu.sync_copy(acc_sh.at[pl.ds(sid * STRIPE, STRIPE)],
                    out_hbm.at[cid, pl.ds(sid * STRIPE, STRIPE)])


RB = 5120
RBP = 5000


def _dinv_block(degp):
    deg = jnp.sum(degp, axis=1) + 1.0
    return lax.rsqrt(deg)


def _tc_pre_body(x_ref, w_ref, degp_ref, g_ref):
    dinv = _dinv_block(degp_ref[...])
    h = jnp.dot(x_ref[...], w_ref[...], preferred_element_type=jnp.float32)
    g_ref[...] = (h * dinv[:, None]).astype(jnp.bfloat16)


def _tc_mid_body(p_ref, degp_ref, b_ref, w_ref, g2_ref):
    dinv = _dinv_block(degp_ref[...])
    s = p_ref[0] + p_ref[1]
    z = jnp.maximum(s * dinv[:, None] + b_ref[...], 0.0)
    h2 = jnp.dot(z, w_ref[...], preferred_element_type=jnp.float32)
    g2_ref[...] = (h2 * dinv[:, None]).astype(jnp.bfloat16)


def _tc_post_body(p_ref, degp_ref, b_ref, out_ref):
    dinv = _dinv_block(degp_ref[...])
    s = p_ref[0] + p_ref[1]
    out_ref[...] = s * dinv[:, None] + b_ref[...]


_x_spec = pl.BlockSpec((RB, D), lambda i: (i, 0))
_w_spec = pl.BlockSpec((D, D), lambda i: (0, 0))
_degp_spec = pl.BlockSpec((RB, NC), lambda i: (i, 0))
_p_spec = pl.BlockSpec((NC, RB, D), lambda i: (0, i, 0))
_b_spec = pl.BlockSpec((1, D), lambda i: (0, 0))

_tc_pre = pl.pallas_call(
    _tc_pre_body, grid=(NP // RB,),
    in_specs=[_x_spec, _w_spec, _degp_spec],
    out_specs=pl.BlockSpec((RB, D), lambda i: (i, 0)),
    out_shape=jax.ShapeDtypeStruct((NP, D), jnp.bfloat16))

_tc_mid = pl.pallas_call(
    _tc_mid_body, grid=(NP // RB,),
    in_specs=[_p_spec, _degp_spec, _b_spec, _w_spec],
    out_specs=pl.BlockSpec((RB, D), lambda i: (i, 0)),
    out_shape=jax.ShapeDtypeStruct((NP, D), jnp.bfloat16))

_tc_post = pl.pallas_call(
    _tc_post_body, grid=(N // RBP,),
    in_specs=[pl.BlockSpec((NC, RBP, D), lambda i: (0, i, 0)),
              pl.BlockSpec((RBP, NC), lambda i: (i, 0)),
              _b_spec],
    out_specs=pl.BlockSpec((RBP, D), lambda i: (i, 0)),
    out_shape=jax.ShapeDtypeStruct((N, D), jnp.float32))


def _as_i32_pairs(g_bf):
    return lax.bitcast_convert_type(g_bf.reshape(NP, D // 2, 2), jnp.int32)


@jax.jit
def kernel(x, edge_index, batch, W1, b1, W2, b2):
    src = edge_index[0]
    dst = edge_index[1]
    loop = jnp.arange(N, dtype=jnp.int32)
    pad = jnp.full((E2 - E - N,), N, jnp.int32)
    src3 = jnp.concatenate([src, loop, pad]).reshape(NW, NCHUNK, B)
    dst3 = jnp.concatenate([dst, loop, pad]).reshape(NW, NCHUNK, B)
    zeros = jnp.zeros((NP, D), jnp.float32)
    xp = jnp.concatenate([x, jnp.zeros((NP - N, D), jnp.float32)])
    W1p = W1[:, _PINV]
    W2p = W2[:, _PINV]
    b1r = b1.reshape(1, D)
    b2r = b2.reshape(1, D)

    degp = _deg_kernel(dst.reshape(NW, (E // NW) // B2, B2)).T
    g1 = _tc_pre(xp, W1p, degp)
    p1 = _msg_kernel(_as_i32_pairs(g1), src3, dst3, zeros)
    g2 = _tc_mid(p1, degp, b1r, W2p)
    p2 = _msg_kernel(_as_i32_pairs(g2), src3, dst3, zeros)
    out = _tc_post(p2, degp, b2r)
    return out


# speedup vs baseline: 2.2580x; 2.1057x over previous
import functools

import jax
import jax.numpy as jnp
from jax import lax
from jax.experimental import pallas as pl
from jax.experimental.pallas import tpu as pltpu
from jax.experimental.pallas import tpu_sc as plsc

N = 10000
D = 128
E = 320000

NC = 2
NS = 16
NW = NC * NS
EPT = E // NW
B = 80
NCHUNK = EPT // B
RPT = N // NS
RB = 5000
GRID = N // RB

_mesh = plsc.VectorSubcoreMesh(core_axis_name="c", subcore_axis_name="s")


NP = 10240
STRIPE = NP // NS
KB2 = 8
B2 = 125
NG2 = EPT // (KB2 * B2)


@functools.partial(
    pl.kernel,
    out_type=jax.ShapeDtypeStruct((NC, NP), jnp.float32),
    mesh=_mesh,
    scratch_types=[
        pltpu.VMEM((KB2, B2), jnp.int32),
        pltpu.VMEM((128,), jnp.float32),
        pltpu.VMEM((STRIPE,), jnp.float32),
        pltpu.VMEM_SHARED((NP,), jnp.float32),
        pltpu.SemaphoreType.DMA,
    ],
)
def _deg_kernel(dst3_hbm, out_hbm, didx_v, ones_v, zbuf_v, deg_sh, dsem):
    cid = lax.axis_index("c")
    sid = lax.axis_index("s")
    wid = sid * NC + cid

    for k in range(STRIPE // 16):
        zbuf_v[pl.ds(k * 16, 16)] = jnp.zeros((16,), jnp.float32)
    for k in range(128 // 16):
        ones_v[pl.ds(k * 16, 16)] = jnp.ones((16,), jnp.float32)

    pltpu.sync_copy(zbuf_v, deg_sh.at[pl.ds(sid * STRIPE, STRIPE)])
    plsc.subcore_barrier()

    def group(f, carry):
        pltpu.sync_copy(dst3_hbm.at[wid, pl.ds(f * KB2, KB2)], didx_v)
        for j in range(KB2):
            pltpu.async_copy(ones_v.at[pl.ds(0, B2)],
                             deg_sh.at[didx_v.at[j]], dsem, add=True)
        for j in range(KB2):
            pltpu.make_async_copy(ones_v.at[pl.ds(0, B2)],
                                  deg_sh.at[didx_v.at[0]], dsem).wait()
        return carry

    lax.fori_loop(0, NG2, group, 0)
    plsc.subcore_barrier()
    pltpu.sync_copy(deg_sh.at[pl.ds(sid * STRIPE, STRIPE)],
                    out_hbm.at[cid, pl.ds(sid * STRIPE, STRIPE)])


NBUF = 4


@functools.partial(
    pl.kernel,
    out_type=jax.ShapeDtypeStruct((NC, N, D), jnp.float32),
    mesh=_mesh,
    scratch_types=[
        pltpu.VMEM((16, B), jnp.int32),
        pltpu.VMEM((16, B), jnp.int32),
        pltpu.VMEM((NBUF, B, D), jnp.float32),
        pltpu.VMEM_SHARED((N, D), jnp.float32),
        pltpu.SemaphoreType.DMA,
        pltpu.SemaphoreType.DMA,
    ],
)
def _msg_kernel(g_hbm, src3_hbm, dst3_hbm, zeros_hbm, out_hbm,
                sidx_g, didx_g, rows_v, acc_sh, gsem, ssem0):
    cid = lax.axis_index("c")
    sid = lax.axis_index("s")
    wid = sid * NC + cid

    def fetch_group(f):
        p = lax.rem(f, 2)

        @pl.when(f < NCHUNK // 8)
        def _():
            pltpu.sync_copy(src3_hbm.at[wid, pl.ds(f * 8, 8)],
                            sidx_g.at[pl.ds(8 * p, 8)])
            pltpu.sync_copy(dst3_hbm.at[wid, pl.ds(f * 8, 8)],
                            didx_g.at[pl.ds(8 * p, 8)])

        @pl.when(f == NCHUNK // 8)
        def _():
            pltpu.sync_copy(src3_hbm.at[wid, pl.ds(120, NCHUNK - 120)],
                            sidx_g.at[pl.ds(8 * p, NCHUNK - 120)])
            pltpu.sync_copy(dst3_hbm.at[wid, pl.ds(120, NCHUNK - 120)],
                            didx_g.at[pl.ds(8 * p, NCHUNK - 120)])

    def idx_row(c):
        return 8 * lax.rem(lax.div(c, 8), 2) + lax.rem(c, 8)

    def gather(c):
        pltpu.async_copy(g_hbm.at[sidx_g.at[idx_row(c)]],
                         rows_v.at[lax.rem(c, NBUF)], gsem)

    def wait_gather(c):
        pltpu.make_async_copy(g_hbm.at[sidx_g.at[0]],
                              rows_v.at[lax.rem(c, NBUF)], gsem).wait()

    def scatter(c, sem):
        pltpu.async_copy(rows_v.at[lax.rem(c, NBUF)],
                         acc_sh.at[didx_g.at[idx_row(c)]], sem, add=True)

    def drain_scatter(sem):
        pltpu.make_async_copy(rows_v.at[0], acc_sh.at[didx_g.at[0]],
                              sem).wait()

    fetch_group(0)
    gather(0)
    gather(1)
    gather(2)

    @pl.when(sid < 15)
    def _():
        pltpu.sync_copy(zeros_hbm.at[pl.ds(sid * STRIPE, STRIPE)],
                        acc_sh.at[pl.ds(sid * STRIPE, STRIPE)])

    @pl.when(sid == 15)
    def _():
        pltpu.sync_copy(zeros_hbm.at[pl.ds(15 * STRIPE, N - 15 * STRIPE)],
                        acc_sh.at[pl.ds(15 * STRIPE, N - 15 * STRIPE)])

    plsc.subcore_barrier()

    def step(c, carry):
        wait_gather(c)

        @pl.when(c >= 1)
        def _():
            drain_scatter(ssem0)

        scatter(c, ssem0)

        @pl.when(c + 3 < NCHUNK)
        def _():
            @pl.when(lax.rem(c + 3, 8) == 0)
            def _():
                fetch_group(lax.div(c + 3, 8))

            gather(c + 3)

        return carry

    lax.fori_loop(0, NCHUNK, step, 0)
    drain_scatter(ssem0)
    plsc.subcore_barrier()

    @pl.when(sid < 15)
    def _():
        pltpu.sync_copy(acc_sh.at[pl.ds(sid * STRIPE, STRIPE)],
                        out_hbm.at[cid, pl.ds(sid * STRIPE, STRIPE)])

    @pl.when(sid == 15)
    def _():
        pltpu.sync_copy(acc_sh.at[pl.ds(15 * STRIPE, N - 15 * STRIPE)],
                        out_hbm.at[cid, pl.ds(15 * STRIPE, N - 15 * STRIPE)])


def _dinv_block(degp):
    deg = jnp.sum(degp, axis=1) + 1.0
    return lax.rsqrt(deg)


def _tc_pre_body(x_ref, w_ref, degp_ref, g_ref):
    dinv = _dinv_block(degp_ref[...])
    h = jnp.dot(x_ref[...], w_ref[...], preferred_element_type=jnp.float32)
    g_ref[...] = h * dinv[:, None]


def _tc_mid_body(p_ref, g1_ref, degp_ref, b_ref, w_ref, g2_ref):
    dinv = _dinv_block(degp_ref[...])
    s = p_ref[0] + p_ref[1] + g1_ref[...]
    z = jnp.maximum(s * dinv[:, None] + b_ref[...], 0.0)
    h2 = jnp.dot(z, w_ref[...], preferred_element_type=jnp.float32)
    g2_ref[...] = h2 * dinv[:, None]


def _tc_post_body(p_ref, g2_ref, degp_ref, b_ref, out_ref):
    dinv = _dinv_block(degp_ref[...])
    s = p_ref[0] + p_ref[1] + g2_ref[...]
    out_ref[...] = s * dinv[:, None] + b_ref[...]


_x_spec = pl.BlockSpec((RB, D), lambda i: (i, 0))
_w_spec = pl.BlockSpec((D, D), lambda i: (0, 0))
_degp_spec = pl.BlockSpec((RB, NC), lambda i: (i, 0))
_p_spec = pl.BlockSpec((NC, RB, D), lambda i: (0, i, 0))
_b_spec = pl.BlockSpec((1, D), lambda i: (0, 0))
_out_spec = pl.BlockSpec((RB, D), lambda i: (i, 0))
_out_shape = jax.ShapeDtypeStruct((N, D), jnp.float32)

_tc_pre = pl.pallas_call(
    _tc_pre_body, grid=(GRID,),
    in_specs=[_x_spec, _w_spec, _degp_spec],
    out_specs=_out_spec, out_shape=_out_shape)

_tc_mid = pl.pallas_call(
    _tc_mid_body, grid=(GRID,),
    in_specs=[_p_spec, _x_spec, _degp_spec, _b_spec, _w_spec],
    out_specs=_out_spec, out_shape=_out_shape)

_tc_post = pl.pallas_call(
    _tc_post_body, grid=(GRID,),
    in_specs=[_p_spec, _x_spec, _degp_spec, _b_spec],
    out_specs=_out_spec, out_shape=_out_shape)


@jax.jit
def kernel(x, edge_index, batch, W1, b1, W2, b2):
    src = edge_index[0]
    dst = edge_index[1]
    src3 = src.reshape(NW, NCHUNK, B)
    dst3 = dst.reshape(NW, NCHUNK, B)
    zeros = jnp.zeros((N, D), jnp.float32)
    b1r = b1.reshape(1, D)
    b2r = b2.reshape(1, D)

    degp = _deg_kernel(dst.reshape(NW, EPT // B2, B2)).T[:N]
    g1 = _tc_pre(x, W1, degp)
    p1 = _msg_kernel(g1, src3, dst3, zeros)
    g2 = _tc_mid(p1, g1, degp, b1r, W2)
    p2 = _msg_kernel(g2, src3, dst3, zeros)
    out = _tc_post(p2, g2, degp, b2r)
    return out
